# Initial kernel scaffold; baseline (speedup 1.0000x reference)
import functools

import jax
import jax.numpy as jnp
from jax import lax
from jax.experimental import pallas as pl
from jax.experimental.pallas import tpu as pltpu
from jax.experimental.pallas import tpu_sc as plsc

NC = 2
NS = 16
LANES = 16


def _mm_body(x_ref, w_ref, b_ref, o_ref, *, relu):
    acc = jnp.dot(x_ref[...], w_ref[...], preferred_element_type=jnp.float32)
    acc = acc + b_ref[0:1, :]
    o_ref[...] = jnp.maximum(acc, 0.0) if relu else acc


def _mm(x, w, b, *, relu=False, bm=2000):
    m, k = x.shape
    n = w.shape[1]
    b8 = jnp.broadcast_to(b.reshape(1, n), (8, n))
    return pl.pallas_call(
        functools.partial(_mm_body, relu=relu),
        grid=(m // bm,),
        in_specs=[
            pl.BlockSpec((bm, k), lambda i: (i, 0)),
            pl.BlockSpec((k, n), lambda i: (0, 0)),
            pl.BlockSpec((8, n), lambda i: (0, 0)),
        ],
        out_specs=pl.BlockSpec((bm, n), lambda i: (i, 0)),
        out_shape=jax.ShapeDtypeStruct((m, n), jnp.float32),
    )(x, w, b8)


def _edge_mega_body(a_ref, We1, be1, We2, be2, LeW, Leb, Wp1c, bp1, e_ref, c_ref, *, L):
    t = jnp.maximum(jnp.dot(a_ref[...], We1[...], preferred_element_type=jnp.float32) + be1[0:1, :], 0.0)
    ea = jnp.maximum(jnp.dot(t, We2[...], preferred_element_type=jnp.float32) + be2[0:1, :], 0.0)
    for l in range(L):
        e_ref[l] = jnp.dot(ea, LeW[l], preferred_element_type=jnp.float32) + Leb[l, 0:1, :]
    c_ref[...] = jnp.dot(ea, Wp1c[...], preferred_element_type=jnp.float32) + bp1[0:1, :]


def _edge_mega(edge_attr, We1, be1, We2, be2, LeW, Leb, Wp1c, bp1, *, be=1000):
    E, de = edge_attr.shape
    L, h, _ = LeW.shape
    be1b = jnp.broadcast_to(be1.reshape(1, h), (8, h))
    be2b = jnp.broadcast_to(be2.reshape(1, h), (8, h))
    bp1b = jnp.broadcast_to(bp1.reshape(1, h), (8, h))
    Lebb = jnp.broadcast_to(Leb[:, None, :], (L, 8, h))
    return pl.pallas_call(
        functools.partial(_edge_mega_body, L=L),
        grid=(E // be,),
        in_specs=[
            pl.BlockSpec((be, de), lambda i: (i, 0)),
            pl.BlockSpec((de, h), lambda i: (0, 0)),
            pl.BlockSpec((8, h), lambda i: (0, 0)),
            pl.BlockSpec((h, h), lambda i: (0, 0)),
            pl.BlockSpec((8, h), lambda i: (0, 0)),
            pl.BlockSpec((L, h, h), lambda i: (0, 0, 0)),
            pl.BlockSpec((L, 8, h), lambda i: (0, 0, 0)),
            pl.BlockSpec((h, h), lambda i: (0, 0)),
            pl.BlockSpec((8, h), lambda i: (0, 0)),
        ],
        out_specs=[
            pl.BlockSpec((L, be, h), lambda i: (0, i, 0)),
            pl.BlockSpec((be, h), lambda i: (i, 0)),
        ],
        out_shape=[
            jax.ShapeDtypeStruct((L, E, h), jnp.float32),
            jax.ShapeDtypeStruct((E, h), jnp.float32),
        ],
    )(edge_attr, We1, be1b, We2, be2b, LeW, Lebb, Wp1c, bp1b)


def _mlp_body(h_ref, p0_ref, p1_ref, W1, b1, W2, b2, o_ref, *, res):
    s = h_ref[...] + p0_ref[...] + p1_ref[...]
    u = jnp.maximum(jnp.dot(s, W1[...], preferred_element_type=jnp.float32) + b1[0:1, :], 0.0)
    v = jnp.maximum(jnp.dot(u, W2[...], preferred_element_type=jnp.float32) + b2[0:1, :], 0.0)
    o_ref[...] = v + h_ref[...] if res else v


def _mlp(h, p2, W1, b1, W2, b2, *, res, bm=2000):
    n_nodes, hd = h.shape
    b1b = jnp.broadcast_to(b1.reshape(1, hd), (8, hd))
    b2b = jnp.broadcast_to(b2.reshape(1, hd), (8, hd))
    nb = n_nodes // bm

    def im0(i):
        return (i, 0)

    def im1(i):
        return (i + nb, 0)

    return pl.pallas_call(
        functools.partial(_mlp_body, res=res),
        grid=(nb,),
        in_specs=[
            pl.BlockSpec((bm, hd), im0),
            pl.BlockSpec((bm, hd), im0),
            pl.BlockSpec((bm, hd), im1),
            pl.BlockSpec((hd, hd), lambda i: (0, 0)),
            pl.BlockSpec((8, hd), lambda i: (0, 0)),
            pl.BlockSpec((hd, hd), lambda i: (0, 0)),
            pl.BlockSpec((8, hd), lambda i: (0, 0)),
        ],
        out_specs=pl.BlockSpec((bm, hd), im0),
        out_shape=jax.ShapeDtypeStruct((n_nodes, hd), jnp.float32),
    )(h, p2, p2, W1, b1b, W2, b2b)


def _sc_aggr_build(N, E, H, CH):
    EPC = E // (NC * NS)
    NCH = EPC // CH
    RPS = N // NS
    ZR = 125
    mesh = plsc.VectorSubcoreMesh(core_axis_name="c", subcore_axis_name="s",
                                  num_cores=NC, num_subcores=NS)

    @functools.partial(
        pl.kernel, mesh=mesh,
        out_type=jax.ShapeDtypeStruct((NC * N, H), jnp.float32),
        scratch_types=[
            pltpu.VMEM_SHARED((N, H), jnp.float32),
            pltpu.VMEM((CH,), jnp.int32),
            pltpu.VMEM((CH,), jnp.int32),
            pltpu.VMEM((CH,), jnp.int32),
            pltpu.VMEM((CH, H), jnp.float32),
            pltpu.VMEM((CH, H), jnp.float32),
            pltpu.VMEM((CH, H), jnp.float32),
            pltpu.VMEM((125, H), jnp.float32),
            pltpu.VMEM((LANES,), jnp.int32),
            pltpu.SemaphoreType.DMA,
        ],
    )
    def aggr(h_hbm, ef_hbm, li_hbm, row_hbm, col_hbm, out_hbm,
             acc_sh, idx_r, idx_c, idx_e, gbuf, ebuf, mbuf, zbuf, livec, sem):
        c = lax.axis_index("c")
        s = lax.axis_index("s")
        base = c * (E // NC) + s * EPC

        def zrow(i, _):
            for k in range(H // LANES):
                zbuf[i, pl.ds(k * LANES, LANES)] = jnp.zeros((LANES,), jnp.float32)
            return 0
        lax.fori_loop(0, ZR, zrow, 0)
        for t in range(RPS // ZR):
            pltpu.sync_copy(zbuf, acc_sh.at[pl.ds(s * RPS + t * ZR, ZR), :])
        pltpu.sync_copy(li_hbm, livec)
        plsc.subcore_barrier()

        iota = lax.iota(jnp.int32, LANES)

        def chunk(ci, _):
            st = base + ci * CH
            pltpu.sync_copy(row_hbm.at[pl.ds(st, CH)], idx_r)
            pltpu.sync_copy(col_hbm.at[pl.ds(st, CH)], idx_c)
            for jj in range(CH // LANES):
                idx_e[pl.ds(jj * LANES, LANES)] = livec[...] * E + (st + jj * LANES) + iota
            pltpu.async_copy(h_hbm.at[idx_r], gbuf, sem).wait()
            pltpu.async_copy(ef_hbm.at[idx_e], ebuf, sem).wait()

            def edge(j, _):
                for k in range(H // LANES):
                    sl = pl.ds(k * LANES, LANES)
                    mbuf[j, sl] = jnp.maximum(gbuf[j, sl] + ebuf[j, sl], 0.0)
                return 0
            lax.fori_loop(0, CH, edge, 0)
            pltpu.sync_copy(mbuf, acc_sh.at[idx_c], add=True)
            return 0
        lax.fori_loop(0, NCH, chunk, 0)

        plsc.subcore_barrier()
        pltpu.sync_copy(acc_sh.at[pl.ds(s * RPS, RPS), :],
                        out_hbm.at[pl.ds(c * N + s * RPS, RPS), :])

    return aggr


def _sc_pred_build(N, E, H, CH):
    EPC = E // (NC * NS)
    NCH = EPC // CH
    mesh = plsc.VectorSubcoreMesh(core_axis_name="c", subcore_axis_name="s",
                                  num_cores=NC, num_subcores=NS)

    @functools.partial(
        pl.kernel, mesh=mesh,
        out_type=jax.ShapeDtypeStruct((E,), jnp.float32),
        scratch_types=[
            pltpu.VMEM((CH,), jnp.int32),
            pltpu.VMEM((CH,), jnp.int32),
            pltpu.VMEM((CH, H), jnp.float32),
            pltpu.VMEM((CH, H), jnp.float32),
            pltpu.VMEM((CH, H), jnp.float32),
            pltpu.VMEM((CH, LANES), jnp.float32),
            pltpu.VMEM((CH,), jnp.float32),
            pltpu.VMEM((H,), jnp.float32),
            pltpu.VMEM((LANES,), jnp.float32),
            pltpu.SemaphoreType.DMA,
        ],
    )
    def pred(a_hbm, b_hbm, c_hbm, row_hbm, col_hbm, w2_hbm, b2_hbm, out_hbm,
             idx_r, idx_c, abuf, bbuf, cbuf, tbuf, pbuf, w2buf, b2buf, sem):
        c = lax.axis_index("c")
        s = lax.axis_index("s")
        base = c * (E // NC) + s * EPC
        pltpu.sync_copy(w2_hbm, w2buf)
        pltpu.sync_copy(b2_hbm, b2buf)
        iota = lax.iota(jnp.int32, LANES)

        def chunk(ci, _):
            st = base + ci * CH
            pltpu.sync_copy(row_hbm.at[pl.ds(st, CH)], idx_r)
            pltpu.sync_copy(col_hbm.at[pl.ds(st, CH)], idx_c)
            pltpu.async_copy(a_hbm.at[idx_r], abuf, sem).wait()
            pltpu.async_copy(b_hbm.at[idx_c], bbuf, sem).wait()
            pltpu.sync_copy(c_hbm.at[pl.ds(st, CH), :], cbuf)

            def edge(j, _):
                acc = jnp.zeros((LANES,), jnp.float32)
                for k in range(H // LANES):
                    sl = pl.ds(k * LANES, LANES)
                    v = jnp.maximum(abuf[j, sl] + bbuf[j, sl] + cbuf[j, sl], 0.0)
                    acc = acc + v * w2buf[sl]
                tbuf[j, :] = acc
                return 0
            lax.fori_loop(0, CH, edge, 0)

            def grp(g, _):
                s16 = b2buf[...]
                for k in range(LANES):
                    s16 = s16 + plsc.load_gather(
                        tbuf, [g * LANES + iota, jnp.full((LANES,), k, jnp.int32)])
                pbuf[pl.ds(g * LANES, LANES)] = s16
                return 0
            lax.fori_loop(0, CH // LANES, grp, 0)
            pltpu.sync_copy(pbuf, out_hbm.at[pl.ds(st, CH)])
            return 0
        lax.fori_loop(0, NCH, chunk, 0)

    return pred


def kernel(x, edge_index, edge_attr, Wn, bn, We1, be1, We2, be2,
           LeW, Leb, L1W, L1b, L2W, L2b, Wp1, bp1, Wp2, bp2):
    N, DIN = x.shape
    E = edge_index.shape[1]
    H = Wn.shape[1]
    L = LeW.shape[0]
    row, col = edge_index[0], edge_index[1]

    h = _mm(x, Wn, bn, relu=False)
    e_all, C = _edge_mega(edge_attr, We1, be1, We2, be2, LeW, Leb,
                          Wp1[2 * H:], bp1)
    ef = e_all.reshape(L * E, H)

    aggr = _sc_aggr_build(N, E, H, CH=80)
    for i in range(0, L, 2):
        li0 = jnp.full((LANES,), i, jnp.int32)
        li1 = jnp.full((LANES,), i + 1, jnp.int32)
        p = aggr(h, ef, li0, row, col)
        x1 = _mlp(h, p, L1W[i], L1b[i], L2W[i], L2b[i], res=False)
        p2 = aggr(x1, ef, li1, row, col)
        h = _mlp(x1, p2, L1W[i + 1], L1b[i + 1], L2W[i + 1], L2b[i + 1], res=True)

    WAB = jnp.concatenate([Wp1[:H], Wp1[H:2 * H]], axis=1)
    ab = _mm(h, WAB, jnp.zeros((2 * H,), jnp.float32), relu=False)
    A = ab[:, :H]
    B = ab[:, H:]

    predk = _sc_pred_build(N, E, H, CH=80)
    b2pad = jnp.broadcast_to(bp2, (LANES,)).astype(jnp.float32)
    pred = predk(A, B, C, row, col, Wp2[:, 0], b2pad)
    return pred


# speedup vs baseline: 2.3784x; 2.3784x over previous
import functools

import jax
import jax.numpy as jnp
from jax import lax
from jax.experimental import pallas as pl
from jax.experimental.pallas import tpu as pltpu
from jax.experimental.pallas import tpu_sc as plsc

NC = 2
NS = 16
LANES = 16


def _mm_body(x_ref, w_ref, b_ref, o_ref, *, relu):
    acc = jnp.dot(x_ref[...], w_ref[...], preferred_element_type=jnp.float32)
    acc = acc + b_ref[0:1, :]
    o_ref[...] = jnp.maximum(acc, 0.0) if relu else acc


def _mm(x, w, b, *, relu=False, bm=2048):
    m, k = x.shape
    n = w.shape[1]
    b8 = jnp.broadcast_to(b.reshape(1, n), (8, n))
    return pl.pallas_call(
        functools.partial(_mm_body, relu=relu),
        grid=(m // bm,),
        in_specs=[
            pl.BlockSpec((bm, k), lambda i: (i, 0)),
            pl.BlockSpec((k, n), lambda i: (0, 0)),
            pl.BlockSpec((8, n), lambda i: (0, 0)),
        ],
        out_specs=pl.BlockSpec((bm, n), lambda i: (i, 0)),
        out_shape=jax.ShapeDtypeStruct((m, n), jnp.float32),
    )(x, w, b8)


def _edge_mega_body(a_ref, We1, be1, We2, be2, LeW, Leb, Wp1c, bp1, e_ref, c_ref, *, L):
    t = jnp.maximum(jnp.dot(a_ref[...], We1[...], preferred_element_type=jnp.float32) + be1[0:1, :], 0.0)
    ea = jnp.maximum(jnp.dot(t, We2[...], preferred_element_type=jnp.float32) + be2[0:1, :], 0.0)
    for l in range(L):
        e_ref[l] = jnp.dot(ea, LeW[l], preferred_element_type=jnp.float32) + Leb[l, 0:1, :]
    c_ref[...] = jnp.dot(ea, Wp1c[...], preferred_element_type=jnp.float32) + bp1[0:1, :]


def _edge_mega(edge_attr, We1, be1, We2, be2, LeW, Leb, Wp1c, bp1, *, be=1000):
    E, de = edge_attr.shape
    L, h, _ = LeW.shape
    be1b = jnp.broadcast_to(be1.reshape(1, h), (8, h))
    be2b = jnp.broadcast_to(be2.reshape(1, h), (8, h))
    bp1b = jnp.broadcast_to(bp1.reshape(1, h), (8, h))
    Lebb = jnp.broadcast_to(Leb[:, None, :], (L, 8, h))
    return pl.pallas_call(
        functools.partial(_edge_mega_body, L=L),
        grid=(E // be,),
        in_specs=[
            pl.BlockSpec((be, de), lambda i: (i, 0)),
            pl.BlockSpec((de, h), lambda i: (0, 0)),
            pl.BlockSpec((8, h), lambda i: (0, 0)),
            pl.BlockSpec((h, h), lambda i: (0, 0)),
            pl.BlockSpec((8, h), lambda i: (0, 0)),
            pl.BlockSpec((L, h, h), lambda i: (0, 0, 0)),
            pl.BlockSpec((L, 8, h), lambda i: (0, 0, 0)),
            pl.BlockSpec((h, h), lambda i: (0, 0)),
            pl.BlockSpec((8, h), lambda i: (0, 0)),
        ],
        out_specs=[
            pl.BlockSpec((L, be, h), lambda i: (0, i, 0)),
            pl.BlockSpec((be, h), lambda i: (i, 0)),
        ],
        out_shape=[
            jax.ShapeDtypeStruct((L, E, h), jnp.float32),
            jax.ShapeDtypeStruct((E, h), jnp.float32),
        ],
    )(edge_attr, We1, be1b, We2, be2b, LeW, Lebb, Wp1c, bp1b)


def _mlp_body(h_ref, p0_ref, p1_ref, W1, b1, W2, b2, o_ref, *, res):
    s = h_ref[...] + p0_ref[...] + p1_ref[...]
    u = jnp.maximum(jnp.dot(s, W1[...], preferred_element_type=jnp.float32) + b1[0:1, :], 0.0)
    v = jnp.maximum(jnp.dot(u, W2[...], preferred_element_type=jnp.float32) + b2[0:1, :], 0.0)
    o_ref[...] = v + h_ref[...] if res else v


def _mlp(h, p2, W1, b1, W2, b2, *, res, bm=2048):
    n_nodes, hd = h.shape
    b1b = jnp.broadcast_to(b1.reshape(1, hd), (8, hd))
    b2b = jnp.broadcast_to(b2.reshape(1, hd), (8, hd))
    nb = n_nodes // bm

    def im0(i):
        return (i, 0)

    def im1(i):
        return (i + nb, 0)

    return pl.pallas_call(
        functools.partial(_mlp_body, res=res),
        grid=(nb,),
        in_specs=[
            pl.BlockSpec((bm, hd), im0),
            pl.BlockSpec((bm, hd), im0),
            pl.BlockSpec((bm, hd), im1),
            pl.BlockSpec((hd, hd), lambda i: (0, 0)),
            pl.BlockSpec((8, hd), lambda i: (0, 0)),
            pl.BlockSpec((hd, hd), lambda i: (0, 0)),
            pl.BlockSpec((8, hd), lambda i: (0, 0)),
        ],
        out_specs=pl.BlockSpec((bm, hd), im0),
        out_shape=jax.ShapeDtypeStruct((n_nodes, hd), jnp.float32),
    )(h, p2, p2, W1, b1b, W2, b2b)


def _sc_aggr_build(N, E, H, CH):
    EPC = E // (NC * NS)
    NCH = EPC // CH
    RPS = N // NS
    ZR = 128
    mesh = plsc.VectorSubcoreMesh(core_axis_name="c", subcore_axis_name="s",
                                  num_cores=NC, num_subcores=NS)

    @functools.partial(
        pl.kernel, mesh=mesh,
        out_type=jax.ShapeDtypeStruct((NC * N, H), jnp.float32),
        scratch_types=[
            pltpu.VMEM_SHARED((N, H), jnp.float32),
            pltpu.VMEM((CH,), jnp.int32),
            pltpu.VMEM((CH,), jnp.int32),
            pltpu.VMEM((CH,), jnp.int32),
            pltpu.VMEM((CH, H), jnp.float32),
            pltpu.VMEM((CH, H), jnp.float32),
            pltpu.VMEM((CH, H), jnp.float32),
            pltpu.VMEM((128, H), jnp.float32),
            pltpu.VMEM((LANES,), jnp.int32),
            pltpu.SemaphoreType.DMA,
        ],
    )
    def aggr(h_hbm, ef_hbm, li_hbm, row_hbm, col_hbm, out_hbm,
             acc_sh, idx_r, idx_c, idx_e, gbuf, ebuf, mbuf, zbuf, livec, sem):
        c = lax.axis_index("c")
        s = lax.axis_index("s")
        base = c * (E // NC) + s * EPC

        def zrow(i, _):
            for k in range(H // LANES):
                zbuf[i, pl.ds(k * LANES, LANES)] = jnp.zeros((LANES,), jnp.float32)
            return 0
        lax.fori_loop(0, ZR, zrow, 0)
        for t in range(RPS // ZR):
            pltpu.sync_copy(zbuf, acc_sh.at[pl.ds(s * RPS + t * ZR, ZR), :])
        pltpu.sync_copy(li_hbm, livec)
        plsc.subcore_barrier()

        iota = lax.iota(jnp.int32, LANES)

        def chunk(ci, _):
            st = base + ci * CH
            pltpu.sync_copy(row_hbm.at[pl.ds(st, CH)], idx_r)
            pltpu.sync_copy(col_hbm.at[pl.ds(st, CH)], idx_c)
            for jj in range(CH // LANES):
                idx_e[pl.ds(jj * LANES, LANES)] = livec[...] * E + (st + jj * LANES) + iota
            pltpu.async_copy(h_hbm.at[idx_r], gbuf, sem).wait()
            pltpu.async_copy(ef_hbm.at[idx_e], ebuf, sem).wait()

            def edge(j, _):
                for k in range(H // LANES):
                    sl = pl.ds(k * LANES, LANES)
                    mbuf[j, sl] = jnp.maximum(gbuf[j, sl] + ebuf[j, sl], 0.0)
                return 0
            lax.fori_loop(0, CH, edge, 0)
            pltpu.sync_copy(mbuf, acc_sh.at[idx_c], add=True)
            return 0
        lax.fori_loop(0, NCH, chunk, 0)

        plsc.subcore_barrier()
        pltpu.sync_copy(acc_sh.at[pl.ds(s * RPS, RPS), :],
                        out_hbm.at[pl.ds(c * N + s * RPS, RPS), :])

    return aggr


def _sc_pred_build(N, E, H, CH):
    EPC = E // (NC * NS)
    NCH = EPC // CH
    mesh = plsc.VectorSubcoreMesh(core_axis_name="c", subcore_axis_name="s",
                                  num_cores=NC, num_subcores=NS)

    @functools.partial(
        pl.kernel, mesh=mesh,
        out_type=jax.ShapeDtypeStruct((E,), jnp.float32),
        scratch_types=[
            pltpu.VMEM((CH,), jnp.int32),
            pltpu.VMEM((CH,), jnp.int32),
            pltpu.VMEM((CH, H), jnp.float32),
            pltpu.VMEM((CH, H), jnp.float32),
            pltpu.VMEM((CH, H), jnp.float32),
            pltpu.VMEM((CH * LANES,), jnp.float32),
            pltpu.VMEM((CH,), jnp.float32),
            pltpu.VMEM((H,), jnp.float32),
            pltpu.VMEM((LANES,), jnp.float32),
            pltpu.SemaphoreType.DMA,
        ],
    )
    def pred(a_hbm, b_hbm, c_hbm, row_hbm, col_hbm, w2_hbm, b2_hbm, out_hbm,
             idx_r, idx_c, abuf, bbuf, cbuf, tbuf, pbuf, w2buf, b2buf, sem):
        c = lax.axis_index("c")
        s = lax.axis_index("s")
        base = c * (E // NC) + s * EPC
        pltpu.sync_copy(w2_hbm, w2buf)
        pltpu.sync_copy(b2_hbm, b2buf)
        iota = lax.iota(jnp.int32, LANES)

        def chunk(ci, _):
            st = base + ci * CH
            pltpu.sync_copy(row_hbm.at[pl.ds(st, CH)], idx_r)
            pltpu.sync_copy(col_hbm.at[pl.ds(st, CH)], idx_c)
            pltpu.async_copy(a_hbm.at[idx_r], abuf, sem).wait()
            pltpu.async_copy(b_hbm.at[idx_c], bbuf, sem).wait()
            pltpu.sync_copy(c_hbm.at[pl.ds(st, CH), :], cbuf)

            def grp(g, _):
                j0 = g * LANES
                s16 = b2buf[...]
                for l in range(LANES):
                    j = j0 + l
                    acc = jnp.zeros((LANES,), jnp.float32)
                    for k in range(H // LANES):
                        sl = pl.ds(k * LANES, LANES)
                        v = jnp.maximum(abuf[j, sl] + bbuf[j, sl] + cbuf[j, sl], 0.0)
                        acc = acc + v * w2buf[sl]
                    for sh in (8, 4, 2, 1):
                        acc = acc + acc.at[iota ^ sh].get(
                            mode="promise_in_bounds")
                    s16 = jnp.where(iota == l, acc + b2buf[...], s16)
                pbuf[pl.ds(j0, LANES)] = s16
                return 0
            lax.fori_loop(0, CH // LANES, grp, 0)
            pltpu.sync_copy(pbuf, out_hbm.at[pl.ds(st, CH)])
            return 0
        lax.fori_loop(0, NCH, chunk, 0)

    return pred


def kernel(x, edge_index, edge_attr, Wn, bn, We1, be1, We2, be2,
           LeW, Leb, L1W, L1b, L2W, L2b, Wp1, bp1, Wp2, bp2):
    N, DIN = x.shape
    E = edge_index.shape[1]
    H = Wn.shape[1]
    L = LeW.shape[0]
    row, col = edge_index[0], edge_index[1]

    NP = ((N + NS * 8 * LANES - 1) // (NS * 8 * LANES)) * (NS * 8 * LANES)
    xp = jnp.pad(x, ((0, NP - N), (0, 0)))

    h = _mm(xp, Wn, bn, relu=False)
    e_all, C = _edge_mega(edge_attr, We1, be1, We2, be2, LeW, Leb,
                          Wp1[2 * H:], bp1)
    ef = e_all.reshape(L * E, H)

    aggr = _sc_aggr_build(NP, E, H, CH=80)
    for i in range(0, L, 2):
        li0 = jnp.full((LANES,), i, jnp.int32)
        li1 = jnp.full((LANES,), i + 1, jnp.int32)
        p = aggr(h, ef, li0, row, col)
        x1 = _mlp(h, p, L1W[i], L1b[i], L2W[i], L2b[i], res=False)
        p2 = aggr(x1, ef, li1, row, col)
        h = _mlp(x1, p2, L1W[i + 1], L1b[i + 1], L2W[i + 1], L2b[i + 1], res=True)

    WAB = jnp.concatenate([Wp1[:H], Wp1[H:2 * H]], axis=1)
    ab = _mm(h, WAB, jnp.zeros((2 * H,), jnp.float32), relu=False)
    A = ab[:, :H]
    B = ab[:, H:]

    predk = _sc_pred_build(NP, E, H, CH=80)
    b2pad = jnp.broadcast_to(bp2, (LANES,)).astype(jnp.float32)
    pred = predk(A, B, C, row, col, Wp2[:, 0], b2pad)
    return pred
